# Initial kernel scaffold; baseline (speedup 1.0000x reference)
import functools

import jax
import jax.numpy as jnp
from jax import lax
from jax.experimental import pallas as pl
from jax.experimental.pallas import tpu as pltpu
from jax.experimental.pallas import tpu_sc as plsc

D_EMBED = 32
NUM_CORES = 2
NUM_SUBCORES = 16
NUM_WORKERS = NUM_CORES * NUM_SUBCORES
CHUNK = 1024


@functools.lru_cache(maxsize=None)
def _make_gather(n_ids: int):
    assert n_ids % (NUM_WORKERS * CHUNK) == 0
    per_worker = n_ids // NUM_WORKERS
    n_chunks = per_worker // CHUNK

    mesh = plsc.VectorSubcoreMesh(core_axis_name="c", subcore_axis_name="s")

    @functools.partial(
        pl.kernel,
        mesh=mesh,
        out_type=jax.ShapeDtypeStruct((n_ids, D_EMBED), jnp.float32),
        scratch_types=[
            pltpu.VMEM((CHUNK,), jnp.int32),
            pltpu.VMEM((CHUNK, D_EMBED), jnp.float32),
            pltpu.SemaphoreType.DMA,
        ],
    )
    def gather_kernel(table_hbm, idx_hbm, out_hbm, idx_v, rows_v, sem):
        wid = lax.axis_index("s") * NUM_CORES + lax.axis_index("c")
        base = wid * per_worker

        def body(g, carry):
            off = base + g * CHUNK
            pltpu.sync_copy(idx_hbm.at[pl.ds(off, CHUNK)], idx_v)
            pltpu.async_copy(table_hbm.at[idx_v], rows_v, sem).wait()
            pltpu.sync_copy(rows_v, out_hbm.at[pl.ds(off, CHUNK)])
            return carry

        lax.fori_loop(0, n_chunks, body, 0)

    return gather_kernel


def kernel(item_ids, table):
    ids = item_ids.astype(jnp.int32).reshape(-1)
    out = _make_gather(ids.shape[0])(table, ids)
    return out.reshape(item_ids.shape + (D_EMBED,))


# speedup vs baseline: 1.4592x; 1.4592x over previous
import functools

import jax
import jax.numpy as jnp
from jax import lax
from jax.experimental import pallas as pl
from jax.experimental.pallas import tpu as pltpu
from jax.experimental.pallas import tpu_sc as plsc

D_EMBED = 32
NUM_CORES = 2
NUM_SUBCORES = 16
NUM_WORKERS = NUM_CORES * NUM_SUBCORES
CHUNK = 1024


@functools.lru_cache(maxsize=None)
def _make_gather(n_ids: int):
    assert n_ids % (NUM_WORKERS * CHUNK) == 0
    per_worker = n_ids // NUM_WORKERS
    n_chunks = per_worker // CHUNK

    mesh = plsc.VectorSubcoreMesh(core_axis_name="c", subcore_axis_name="s")

    @functools.partial(
        pl.kernel,
        mesh=mesh,
        out_type=jax.ShapeDtypeStruct((n_ids, D_EMBED), jnp.float32),
        scratch_types=[
            pltpu.VMEM((CHUNK,), jnp.int32),
            pltpu.VMEM((CHUNK, D_EMBED), jnp.float32),
            pltpu.SemaphoreType.DMA,
        ],
        compiler_params=pltpu.CompilerParams(use_tc_tiling_on_sc=False),
    )
    def gather_kernel(table_hbm, idx_hbm, out_hbm, idx_v, rows_v, sem):
        wid = lax.axis_index("s") * NUM_CORES + lax.axis_index("c")
        base = wid * per_worker

        def body(g, carry):
            off = base + g * CHUNK
            pltpu.sync_copy(idx_hbm.at[pl.ds(off, CHUNK)], idx_v)
            pltpu.async_copy(table_hbm.at[idx_v], rows_v, sem).wait()
            pltpu.sync_copy(rows_v, out_hbm.at[pl.ds(off, CHUNK)])
            return carry

        lax.fori_loop(0, n_chunks, body, 0)

    return gather_kernel


def kernel(item_ids, table):
    ids = item_ids.astype(jnp.int32).reshape(-1)
    out = _make_gather(ids.shape[0])(table, ids)
    return out.reshape(item_ids.shape + (D_EMBED,))


# speedup vs baseline: 1.5006x; 1.0284x over previous
import functools

import jax
import jax.numpy as jnp
from jax import lax
from jax.experimental import pallas as pl
from jax.experimental.pallas import tpu as pltpu
from jax.experimental.pallas import tpu_sc as plsc

D_EMBED = 32
NUM_CORES = 2
NUM_SUBCORES = 16
NUM_WORKERS = NUM_CORES * NUM_SUBCORES
CHUNK = 1280
NBUF = 2


@functools.lru_cache(maxsize=None)
def _make_gather(n_ids: int):
    assert n_ids % (NUM_WORKERS * CHUNK * NBUF) == 0
    per_worker = n_ids // NUM_WORKERS
    n_chunks = per_worker // CHUNK

    mesh = plsc.VectorSubcoreMesh(core_axis_name="c", subcore_axis_name="s")

    @functools.partial(
        pl.kernel,
        mesh=mesh,
        out_type=jax.ShapeDtypeStruct((n_ids, D_EMBED), jnp.float32),
        scratch_types=[
            pltpu.VMEM((per_worker,), jnp.int32),
            pltpu.VMEM((NBUF, CHUNK, D_EMBED), jnp.float32),
            [pltpu.SemaphoreType.DMA] * NBUF,
        ],
        compiler_params=pltpu.CompilerParams(use_tc_tiling_on_sc=False),
    )
    def gather_kernel(table_hbm, idx_hbm, out_hbm, idx_v, rows_v, sems):
        wid = lax.axis_index("s") * NUM_CORES + lax.axis_index("c")
        base = wid * per_worker
        pltpu.sync_copy(idx_hbm.at[pl.ds(base, per_worker)], idx_v)

        def gather_desc(g, p):
            idx_chunk = idx_v.at[pl.ds(g * CHUNK, CHUNK)]
            return pltpu.make_async_copy(
                table_hbm.at[idx_chunk], rows_v.at[p], sems[p]
            )

        for p in range(NBUF):
            gather_desc(p, p).start()

        @pl.loop(0, n_chunks // NBUF - 1)
        def _(k):
            for p in range(NBUF):
                g = k * NBUF + p
                gather_desc(g, p).wait()
                pltpu.sync_copy(
                    rows_v.at[p], out_hbm.at[pl.ds(base + g * CHUNK, CHUNK)]
                )
                gather_desc(g + NBUF, p).start()

        for p in range(NBUF):
            g = n_chunks - NBUF + p
            gather_desc(g, p).wait()
            pltpu.sync_copy(
                rows_v.at[p], out_hbm.at[pl.ds(base + g * CHUNK, CHUNK)]
            )

    return gather_kernel


def kernel(item_ids, table):
    ids = item_ids.astype(jnp.int32).reshape(-1)
    out = _make_gather(ids.shape[0])(table, ids)
    return out.reshape(item_ids.shape + (D_EMBED,))
